# baseline (speedup 1.0000x reference)
import functools

import jax
import jax.numpy as jnp
from jax import lax
from jax.experimental import pallas as pl
from jax.experimental.pallas import tpu as pltpu
from jax.experimental.pallas import tpu_sc as plsc

N_NODES = 100000
N_LINKS = 3200000

GRAVITY = 9.81
WATER_DENSITY = 1000.0
ICE_DENSITY = 917.0
LATENT_HEAT = 335000.0
STEP_HEIGHT = 0.1
ICE_FLUIDITY = 6e-24
GLENS_N = 3
MELT_CONSTANT = 1.0 / (ICE_DENSITY * LATENT_HEAT)
CLOSURE_CONSTANT = 2.0 * ICE_FLUIDITY * GLENS_N ** (-GLENS_N)
PHI_COEFF = WATER_DENSITY * GRAVITY

NC = 2
NS = 16
NW = NC * NS
L = 16

LINKS_PER_W = N_LINKS // NW
CHUNK = 2000
NCHUNK = LINKS_PER_W // CHUNK
CHUNK3 = 2000
NCHUNK3 = LINKS_PER_W // CHUNK3

_mesh = plsc.VectorSubcoreMesh(core_axis_name="c", subcore_axis_name="s")
_sc_params = pltpu.CompilerParams(needs_layout_passes=False)


def _worker_id():
    return lax.axis_index("s") * NC + lax.axis_index("c")


@functools.partial(
    pl.kernel,
    out_type=(
        jax.ShapeDtypeStruct((NW, N_NODES), jnp.float32),
        jax.ShapeDtypeStruct((N_LINKS,), jnp.int32),
    ),
    mesh=_mesh,
    compiler_params=_sc_params,
    scratch_types=(
        [pltpu.VMEM((N_NODES,), jnp.float32)]
        + [pltpu.VMEM((CHUNK,), jnp.int32)] * 4
        + [pltpu.VMEM((CHUNK,), jnp.float32)] * 8
        + [pltpu.VMEM((CHUNK,), jnp.int32)] * 2
        + [pltpu.SemaphoreType.DMA] * 4
    ),
)
def _k1(head_hbm, tail_hbm, flux_hbm, slide_hbm, area_hbm, neff_hbm,
        part_hbm, pd_hbm, node_buf, *scr):
    wid = _worker_id()
    base = wid * LINKS_PER_W
    S = 2
    hb = scr[0:S]
    tb = scr[S:2 * S]
    qb = scr[2 * S:3 * S]
    ub = scr[3 * S:4 * S]
    ab = scr[4 * S:5 * S]
    nb_ = scr[5 * S:6 * S]
    ob = scr[6 * S:7 * S]
    isems = scr[7 * S:8 * S]
    osems = scr[8 * S:9 * S]
    himask = jnp.int32(-65536)
    half = jnp.int32(32768)

    def in6(i, b):
        off = base + i * CHUNK
        for hbm, buf in ((head_hbm, hb[b]), (tail_hbm, tb[b]),
                         (flux_hbm, qb[b]), (slide_hbm, ub[b]),
                         (area_hbm, ab[b]), (neff_hbm, nb_[b])):
            pltpu.async_copy(hbm.at[pl.ds(off, CHUNK)], buf, isems[b])

    def wait_in6(i, b):
        off = base + i * CHUNK
        for hbm, buf in ((head_hbm, hb[b]), (tail_hbm, tb[b]),
                         (flux_hbm, qb[b]), (slide_hbm, ub[b]),
                         (area_hbm, ab[b]), (neff_hbm, nb_[b])):
            pltpu.make_async_copy(hbm.at[pl.ds(off, CHUNK)], buf,
                                  isems[b]).wait()

    def out1(i, b):
        off = base + i * CHUNK
        pltpu.async_copy(ob[b], pd_hbm.at[pl.ds(off, CHUNK)], osems[b])

    def wait_out1(i, b):
        off = base + i * CHUNK
        pltpu.make_async_copy(ob[b], pd_hbm.at[pl.ds(off, CHUNK)],
                              osems[b]).wait()

    in6(0, 0)
    zeros = jnp.zeros((L,), jnp.float32)

    @plsc.parallel_loop(0, N_NODES, step=L, unroll=5)
    def zero(o):
        node_buf[pl.ds(o, L)] = zeros

    def chunk_scatter(k, carry):
        for b in range(S):
            i = S * k + b
            nxt = i + 1

            @pl.when(nxt < NCHUNK)
            def _():
                in6(nxt, 1 - b)

            wait_in6(i, b)

            @pl.when(i >= S)
            def _():
                wait_out1(i - S, b)

            @plsc.parallel_loop(0, CHUNK, step=L, unroll=5)
            def vec(o):
                sl = pl.ds(o, L)
                h = hb[b][sl]
                t = tb[b][sl]
                q = qb[b][sl]
                u = ub[b][sl]
                a = ab[b][sl]
                n = nb_[b][sl]
                plsc.addupdate_scatter(node_buf, [h], q)
                plsc.addupdate_scatter(node_buf, [t], -q)
                dense = STEP_HEIGHT * u - CLOSURE_CONSTANT * (n * n * n) * a
                db = plsc.bitcast(dense, jnp.int32)
                qm = plsc.bitcast(MELT_CONSTANT * q, jnp.int32)
                ob[b][sl] = ((db + half) & himask) | lax.shift_right_logical(
                    qm + half, 16)

            out1(i, b)
        return carry

    lax.fori_loop(0, NCHUNK // S, chunk_scatter, 0)
    wait_out1(NCHUNK - 2, 0)
    wait_out1(NCHUNK - 1, 1)
    pltpu.sync_copy(node_buf, part_hbm.at[wid])


def _pack_body(p_ref, m_ref, b_ref, w_ref, z_ref):
    nb = jnp.sum(p_ref[...], axis=0) + m_ref[...]
    phi = PHI_COEFF * b_ref[...] + w_ref[...]
    phi_u = lax.bitcast_convert_type(
        phi.astype(jnp.bfloat16), jnp.uint16).astype(jnp.uint32)
    psi_u = lax.bitcast_convert_type(
        (0.5 * nb).astype(jnp.bfloat16), jnp.uint16).astype(jnp.uint32)
    z_ref[...] = ((phi_u << 16) | psi_u).astype(jnp.int32)


def _pack_tc(partials, meltwater, bedrock, pressure):
    p3 = partials.reshape(NW, 8, N_NODES // 8)
    m2 = meltwater.reshape(8, N_NODES // 8)
    b2 = bedrock.reshape(8, N_NODES // 8)
    w2 = pressure.reshape(8, N_NODES // 8)
    out = pl.pallas_call(
        _pack_body,
        out_shape=jax.ShapeDtypeStruct((8, N_NODES // 8), jnp.int32),
    )(p3, m2, b2, w2)
    return out.reshape(N_NODES)


@functools.partial(
    pl.kernel,
    out_type=jax.ShapeDtypeStruct((N_LINKS,), jnp.float32),
    mesh=_mesh,
    compiler_params=_sc_params,
    scratch_types=(
        [pltpu.VMEM((N_NODES,), jnp.int32)]
        + [pltpu.VMEM((CHUNK3,), jnp.int32)] * 6
        + [pltpu.VMEM((CHUNK3,), jnp.float32)] * 2
        + [pltpu.SemaphoreType.DMA] * 4
    ),
)
def _k3(z_hbm, head_hbm, tail_hbm, pd_hbm, out_hbm, node_buf, *scr):
    wid = _worker_id()
    base = wid * LINKS_PER_W
    S = 2
    hb = scr[0:S]
    tb = scr[S:2 * S]
    pb = scr[2 * S:3 * S]
    ob = scr[3 * S:4 * S]
    isems = scr[4 * S:5 * S]
    osems = scr[5 * S:6 * S]
    himask = jnp.int32(-65536)

    def in3(i, b):
        off = base + i * CHUNK3
        for hbm, buf in ((head_hbm, hb[b]), (tail_hbm, tb[b]),
                         (pd_hbm, pb[b])):
            pltpu.async_copy(hbm.at[pl.ds(off, CHUNK3)], buf, isems[b])

    def wait_in3(i, b):
        off = base + i * CHUNK3
        for hbm, buf in ((head_hbm, hb[b]), (tail_hbm, tb[b]),
                         (pd_hbm, pb[b])):
            pltpu.make_async_copy(hbm.at[pl.ds(off, CHUNK3)], buf,
                                  isems[b]).wait()

    def out1(i, b):
        off = base + i * CHUNK3
        pltpu.async_copy(ob[b], out_hbm.at[pl.ds(off, CHUNK3)], osems[b])

    def wait_out1(i, b):
        off = base + i * CHUNK3
        pltpu.make_async_copy(ob[b], out_hbm.at[pl.ds(off, CHUNK3)],
                              osems[b]).wait()

    in3(0, 0)
    pltpu.sync_copy(z_hbm, node_buf)

    def chunk(k, carry):
        for b in range(S):
            i = S * k + b
            nxt = i + 1

            @pl.when(nxt < NCHUNK3)
            def _():
                in3(nxt, 1 - b)

            wait_in3(i, b)

            @pl.when(i >= S)
            def _():
                wait_out1(i - S, b)

            @plsc.parallel_loop(0, CHUNK3, step=L, unroll=5)
            def vec(o):
                sl = pl.ds(o, L)
                h = hb[b][sl]
                t = tb[b][sl]
                pd = pb[b][sl]
                zh = plsc.load_gather(node_buf, [h])
                zt = plsc.load_gather(node_buf, [t])
                phi_h = plsc.bitcast(zh & himask, jnp.float32)
                phi_t = plsc.bitcast(zt & himask, jnp.float32)
                psi = plsc.bitcast(zh << 16, jnp.float32) + plsc.bitcast(
                    zt << 16, jnp.float32)
                dense = plsc.bitcast(pd & himask, jnp.float32)
                qm = plsc.bitcast(pd << 16, jnp.float32)
                ob[b][sl] = qm * (phi_t - phi_h) + dense + psi

            out1(i, b)
        return carry

    lax.fori_loop(0, NCHUNK3 // S, chunk, 0)
    wait_out1(NCHUNK3 - 2, 0)
    wait_out1(NCHUNK3 - 1, 1)


def kernel(bedrock_elevation, ice_thickness, meltwater_input, water_pressure,
           ice_sliding_velocity, conduit_area, effective_pressure, water_flux,
           node_at_link_head, node_at_link_tail):
    del ice_thickness
    head = node_at_link_head.astype(jnp.int32)
    tail = node_at_link_tail.astype(jnp.int32)
    partials, pd = _k1(head, tail, water_flux, ice_sliding_velocity,
                       conduit_area, effective_pressure)
    z = _pack_tc(partials, meltwater_input, bedrock_elevation, water_pressure)
    return _k3(z, head, tail, pd)


# speedup vs baseline: 1.0766x; 1.0766x over previous
import functools

import jax
import jax.numpy as jnp
from jax import lax
from jax.experimental import pallas as pl
from jax.experimental.pallas import tpu as pltpu
from jax.experimental.pallas import tpu_sc as plsc

N_NODES = 100000
N_LINKS = 3200000

GRAVITY = 9.81
WATER_DENSITY = 1000.0
ICE_DENSITY = 917.0
LATENT_HEAT = 335000.0
STEP_HEIGHT = 0.1
ICE_FLUIDITY = 6e-24
GLENS_N = 3
MELT_CONSTANT = 1.0 / (ICE_DENSITY * LATENT_HEAT)
CLOSURE_CONSTANT = 2.0 * ICE_FLUIDITY * GLENS_N ** (-GLENS_N)
PHI_COEFF = WATER_DENSITY * GRAVITY

NC = 2
NS = 16
NW = NC * NS
L = 16

LINKS_PER_W = N_LINKS // NW
CHUNK = 2000
NCHUNK = LINKS_PER_W // CHUNK
CHUNK3 = 2000
NCHUNK3 = LINKS_PER_W // CHUNK3

_mesh = plsc.VectorSubcoreMesh(core_axis_name="c", subcore_axis_name="s")
_sc_params = pltpu.CompilerParams(needs_layout_passes=False)


def _worker_id():
    return lax.axis_index("s") * NC + lax.axis_index("c")


@functools.partial(
    pl.kernel,
    out_type=jax.ShapeDtypeStruct((NW, N_NODES), jnp.float32),
    mesh=_mesh,
    compiler_params=_sc_params,
    scratch_types=(
        [pltpu.VMEM((N_NODES,), jnp.float32)]
        + [pltpu.VMEM((CHUNK,), jnp.int32)] * 10
        + [pltpu.VMEM((CHUNK,), jnp.float32)] * 5
        + [pltpu.SemaphoreType.DMA] * 5
    ),
)
def _k1(head_hbm, tail_hbm, flux_hbm, part_hbm, node_buf, *scr):
    wid = _worker_id()
    base = wid * LINKS_PER_W
    S = 5
    hb = scr[0:S]
    tb = scr[S:2 * S]
    qb = scr[2 * S:3 * S]
    isems = scr[3 * S:4 * S]

    def in3(i, b):
        off = base + i * CHUNK
        for hbm, buf in ((head_hbm, hb[b]), (tail_hbm, tb[b]),
                         (flux_hbm, qb[b])):
            pltpu.async_copy(hbm.at[pl.ds(off, CHUNK)], buf, isems[b])

    def wait_in3(i, b):
        off = base + i * CHUNK
        for hbm, buf in ((head_hbm, hb[b]), (tail_hbm, tb[b]),
                         (flux_hbm, qb[b])):
            pltpu.make_async_copy(hbm.at[pl.ds(off, CHUNK)], buf,
                                  isems[b]).wait()

    for j in range(S - 1):
        in3(j, j)
    zeros = jnp.zeros((L,), jnp.float32)

    @plsc.parallel_loop(0, N_NODES, step=L, unroll=5)
    def zero(o):
        node_buf[pl.ds(o, L)] = zeros

    def chunk_scatter(k, carry):
        for b in range(S):
            i = S * k + b
            nxt = i + S - 1

            @pl.when(nxt < NCHUNK)
            def _():
                in3(nxt, (b + S - 1) % S)

            wait_in3(i, b)

            @plsc.parallel_loop(0, CHUNK, step=L, unroll=5)
            def vec(o):
                sl = pl.ds(o, L)
                h = hb[b][sl]
                t = tb[b][sl]
                q = qb[b][sl]
                plsc.addupdate_scatter(node_buf, [h], q)
                plsc.addupdate_scatter(node_buf, [t], -q)

        return carry

    lax.fori_loop(0, NCHUNK // S, chunk_scatter, 0)
    pltpu.sync_copy(node_buf, part_hbm.at[wid])


def _pack_body(p_ref, m_ref, b_ref, w_ref, z_ref):
    nb = jnp.sum(p_ref[...], axis=0) + m_ref[...]
    phi = PHI_COEFF * b_ref[...] + w_ref[...]
    phi_u = lax.bitcast_convert_type(
        phi.astype(jnp.bfloat16), jnp.uint16).astype(jnp.uint32)
    psi_u = lax.bitcast_convert_type(
        (0.5 * nb).astype(jnp.bfloat16), jnp.uint16).astype(jnp.uint32)
    z_ref[...] = ((phi_u << 16) | psi_u).astype(jnp.int32)


def _pack_tc(partials, meltwater, bedrock, pressure):
    p3 = partials.reshape(NW, 8, N_NODES // 8)
    m2 = meltwater.reshape(8, N_NODES // 8)
    b2 = bedrock.reshape(8, N_NODES // 8)
    w2 = pressure.reshape(8, N_NODES // 8)
    out = pl.pallas_call(
        _pack_body,
        out_shape=jax.ShapeDtypeStruct((8, N_NODES // 8), jnp.int32),
    )(p3, m2, b2, w2)
    return out.reshape(N_NODES)


@functools.partial(
    pl.kernel,
    out_type=jax.ShapeDtypeStruct((N_LINKS,), jnp.float32),
    mesh=_mesh,
    compiler_params=_sc_params,
    scratch_types=(
        [pltpu.VMEM((N_NODES,), jnp.int32)]
        + [pltpu.VMEM((CHUNK3,), jnp.int32)] * 4
        + [pltpu.VMEM((CHUNK3,), jnp.float32)] * 10
        + [pltpu.SemaphoreType.DMA] * 4
    ),
)
def _k3(z_hbm, head_hbm, tail_hbm, flux_hbm, slide_hbm, area_hbm, neff_hbm,
        out_hbm, node_buf, *scr):
    wid = _worker_id()
    base = wid * LINKS_PER_W
    S = 2
    hb = scr[0:S]
    tb = scr[S:2 * S]
    qb = scr[2 * S:3 * S]
    ub = scr[3 * S:4 * S]
    ab = scr[4 * S:5 * S]
    nb_ = scr[5 * S:6 * S]
    ob = scr[6 * S:7 * S]
    isems = scr[7 * S:8 * S]
    osems = scr[8 * S:9 * S]
    himask = jnp.int32(-65536)

    def in6(i, b):
        off = base + i * CHUNK3
        for hbm, buf in ((head_hbm, hb[b]), (tail_hbm, tb[b]),
                         (flux_hbm, qb[b]), (slide_hbm, ub[b]),
                         (area_hbm, ab[b]), (neff_hbm, nb_[b])):
            pltpu.async_copy(hbm.at[pl.ds(off, CHUNK3)], buf, isems[b])

    def wait_in6(i, b):
        off = base + i * CHUNK3
        for hbm, buf in ((head_hbm, hb[b]), (tail_hbm, tb[b]),
                         (flux_hbm, qb[b]), (slide_hbm, ub[b]),
                         (area_hbm, ab[b]), (neff_hbm, nb_[b])):
            pltpu.make_async_copy(hbm.at[pl.ds(off, CHUNK3)], buf,
                                  isems[b]).wait()

    def out1(i, b):
        off = base + i * CHUNK3
        pltpu.async_copy(ob[b], out_hbm.at[pl.ds(off, CHUNK3)], osems[b])

    def wait_out1(i, b):
        off = base + i * CHUNK3
        pltpu.make_async_copy(ob[b], out_hbm.at[pl.ds(off, CHUNK3)],
                              osems[b]).wait()

    for j in range(S - 1):
        in6(j, j)
    pltpu.sync_copy(z_hbm, node_buf)

    def chunk(k, carry):
        for b in range(S):
            i = S * k + b
            nxt = i + S - 1

            @pl.when(nxt < NCHUNK3)
            def _():
                in6(nxt, (b + S - 1) % S)

            wait_in6(i, b)

            @pl.when(i >= S)
            def _():
                wait_out1(i - S, b)

            @plsc.parallel_loop(0, CHUNK3, step=L, unroll=5)
            def vec(o):
                sl = pl.ds(o, L)
                h = hb[b][sl]
                t = tb[b][sl]
                q = qb[b][sl]
                u = ub[b][sl]
                a = ab[b][sl]
                n = nb_[b][sl]
                zh = plsc.load_gather(node_buf, [h])
                zt = plsc.load_gather(node_buf, [t])
                phi_h = plsc.bitcast(zh & himask, jnp.float32)
                phi_t = plsc.bitcast(zt & himask, jnp.float32)
                psi = plsc.bitcast(zh << 16, jnp.float32) + plsc.bitcast(
                    zt << 16, jnp.float32)
                ob[b][sl] = (MELT_CONSTANT * q * (phi_t - phi_h)
                             + STEP_HEIGHT * u
                             - CLOSURE_CONSTANT * (n * n * n) * a
                             + psi)

            out1(i, b)
        return carry

    lax.fori_loop(0, NCHUNK3 // S, chunk, 0)
    wait_out1(NCHUNK3 - 2, 0)
    wait_out1(NCHUNK3 - 1, 1)


def kernel(bedrock_elevation, ice_thickness, meltwater_input, water_pressure,
           ice_sliding_velocity, conduit_area, effective_pressure, water_flux,
           node_at_link_head, node_at_link_tail):
    del ice_thickness
    head = node_at_link_head.astype(jnp.int32)
    tail = node_at_link_tail.astype(jnp.int32)
    partials = _k1(head, tail, water_flux)
    z = _pack_tc(partials, meltwater_input, bedrock_elevation, water_pressure)
    return _k3(z, head, tail, water_flux, ice_sliding_velocity,
               conduit_area, effective_pressure)
